# baseline (speedup 1.0000x reference)
import functools

import jax
import jax.numpy as jnp
from jax import lax
from jax.experimental import pallas as pl
from jax.experimental.pallas import tpu as pltpu
from jax.experimental.pallas import tpu_sc as plsc

N = 10000
E = 320000
D = 128
G = 64
C = 10

NC = 2
NS = 16
NW = NC * NS

EROWS = 2560
EP = EROWS * 128
ROWS_PER_W = EROWS // NW

PAD_ROWS = 240
ACC_N = N + PAD_ROWS


def _deg_body(dst_hbm, out_hbm, idx_v, hist_v):
  c = lax.axis_index("c")
  s = lax.axis_index("s")
  w = s * NC + c
  pltpu.sync_copy(dst_hbm.at[pl.ds(w * ROWS_PER_W, ROWS_PER_W)], idx_v)
  zeros16 = jnp.zeros((16,), jnp.float32)

  def zbody(i, carry):
    hist_v[pl.ds(i * 16, 16)] = zeros16
    return carry

  lax.fori_loop(0, ACC_N // 16, zbody, 0)
  ones16 = jnp.full((16,), 1.0, jnp.float32)

  def body(r, carry):
    for j in range(8):
      idx = idx_v[r, pl.ds(j * 16, 16)]
      plsc.addupdate_scatter(hist_v, [idx], ones16)
    return carry

  lax.fori_loop(0, ROWS_PER_W, body, 0)
  pltpu.sync_copy(hist_v, out_hbm.at[w])


def _agg_body(z_hbm, p_hbm, src_hbm, dst_hbm, out_hbm, acc, sblk, dblk, rows0,
              rows1, gsem0, gsem1, ssem0, ssem1):
  c = lax.axis_index("c")
  s = lax.axis_index("s")
  w = s * NC + c

  tile_rows = ACC_N // NS
  zbase = s * tile_rows
  pltpu.sync_copy(z_hbm, acc.at[pl.ds(zbase, tile_rows)])
  plsc.subcore_barrier()

  half = ROWS_PER_W // 2
  for h in range(2):
    base = w * ROWS_PER_W + h * half
    pltpu.sync_copy(src_hbm.at[pl.ds(base, half)], sblk)
    pltpu.sync_copy(dst_hbm.at[pl.ds(base, half)], dblk)
    pltpu.async_copy(p_hbm.at[sblk.at[0]], rows0, gsem0)
    pltpu.async_copy(p_hbm.at[sblk.at[1]], rows1, gsem1)

    def ebody(i, carry):
      c0 = 2 * i
      pltpu.make_async_copy(p_hbm.at[sblk.at[c0]], rows0, gsem0).wait()
      pltpu.async_copy(rows0, acc.at[dblk.at[c0]], ssem0, add=True)
      pltpu.make_async_copy(p_hbm.at[sblk.at[c0 + 1]], rows1, gsem1).wait()
      pltpu.async_copy(rows1, acc.at[dblk.at[c0 + 1]], ssem1, add=True)
      pltpu.make_async_copy(rows0, acc.at[dblk.at[c0]], ssem0).wait()
      pltpu.async_copy(p_hbm.at[sblk.at[c0 + 2]], rows0, gsem0)
      pltpu.make_async_copy(rows1, acc.at[dblk.at[c0 + 1]], ssem1).wait()
      pltpu.async_copy(p_hbm.at[sblk.at[c0 + 3]], rows1, gsem1)
      return carry

    lax.fori_loop(0, half // 2 - 1, ebody, 0)
    cl = half - 2
    pltpu.make_async_copy(p_hbm.at[sblk.at[cl]], rows0, gsem0).wait()
    pltpu.sync_copy(rows0, acc.at[dblk.at[cl]], add=True)
    pltpu.make_async_copy(p_hbm.at[sblk.at[cl + 1]], rows1, gsem1).wait()
    pltpu.sync_copy(rows1, acc.at[dblk.at[cl + 1]], add=True)
  plsc.subcore_barrier()

  pltpu.sync_copy(acc.at[pl.ds(zbase, tile_rows)],
                  out_hbm.at[c, pl.ds(zbase, tile_rows)])


@functools.cache
def _sc_kernels():
  mesh = plsc.VectorSubcoreMesh(core_axis_name="c", subcore_axis_name="s",
                                num_cores=NC, num_subcores=NS)
  deg = pl.kernel(
      _deg_body,
      out_type=jax.ShapeDtypeStruct((NW, ACC_N), jnp.float32),
      mesh=mesh,
      scratch_types=[
          pltpu.VMEM((ROWS_PER_W, 128), jnp.int32),
          pltpu.VMEM((ACC_N,), jnp.float32),
      ],
      compiler_params=pltpu.CompilerParams(needs_layout_passes=False),
  )
  agg = pl.kernel(
      _agg_body,
      out_type=jax.ShapeDtypeStruct((NC, ACC_N, 128), jnp.float32),
      mesh=mesh,
      scratch_types=[
          pltpu.VMEM_SHARED((ACC_N, 128), jnp.float32),
          pltpu.VMEM((ROWS_PER_W // 2, 128), jnp.int32),
          pltpu.VMEM((ROWS_PER_W // 2, 128), jnp.int32),
          pltpu.VMEM((128, 128), jnp.float32),
          pltpu.VMEM((128, 128), jnp.float32),
          pltpu.SemaphoreType.DMA,
          pltpu.SemaphoreType.DMA,
          pltpu.SemaphoreType.DMA,
          pltpu.SemaphoreType.DMA,
      ],
  )
  return deg, agg


def _prep_body(parts_ref, x_ref, w1_ref, p_ref, dis_ref):
  deg = jnp.sum(parts_ref[...], axis=1, keepdims=True)[:N] + 1.0
  dis = lax.rsqrt(deg)
  h = jnp.dot(x_ref[...], w1_ref[...], preferred_element_type=jnp.float32,
              precision=lax.Precision.HIGHEST)
  p_ref[...] = h * dis
  dis_ref[...] = dis


def _mid_body(parts_ref, p_ref, dis_ref, kc_ref, w_ref, pn_ref):
  dis = dis_ref[...]
  agg = p_ref[...] + parts_ref[0][:N] + parts_ref[1][:N]
  t = jnp.maximum(dis * agg * kc_ref[0:1] + kc_ref[1:2], 0.0)
  h = jnp.dot(t, w_ref[...], preferred_element_type=jnp.float32,
              precision=lax.Precision.HIGHEST)
  pn_ref[...] = h * dis


def _final_body(parts_ref, p_ref, dis_ref, cc_ref, batch_ref, w2_ref, b2_ref,
                out_ref):
  dis = dis_ref[...]
  agg = p_ref[...] + parts_ref[0][:N] + parts_ref[1][:N]
  h = jnp.maximum(dis * agg + cc_ref[...], 0.0)
  gids = lax.broadcasted_iota(jnp.int32, (G, N), 0)
  mask = (gids == batch_ref[...]).astype(jnp.float32)
  sums = jnp.dot(mask, h, preferred_element_type=jnp.float32,
                 precision=lax.Precision.HIGHEST)
  counts = jnp.sum(mask, axis=1, keepdims=True)
  pooled = sums / jnp.maximum(counts, 1.0)
  logits = jnp.dot(pooled, w2_ref[...], preferred_element_type=jnp.float32,
                   precision=lax.Precision.HIGHEST) + b2_ref[...]
  m = jnp.max(logits, axis=1, keepdims=True)
  e = jnp.exp(logits - m)
  lse = jnp.log(jnp.sum(e, axis=1, keepdims=True))
  out_ref[...] = (logits - m - lse)[:, :C]


def _tc(body, out_shape, *args):
  return pl.pallas_call(body, out_shape=out_shape)(*args)


def kernel(x, edge_index, batch, W1, b1, gamma, beta, running_mean, running_var,
           Wc0, bc0, Wc1, bc1, W2, b2):
  f32 = jnp.float32
  src = edge_index[0].astype(jnp.int32)
  dst = edge_index[1].astype(jnp.int32)
  pad = EP - E
  pad_i = jnp.arange(pad, dtype=jnp.int32)
  src2d = jnp.concatenate([src, pad_i % N]).reshape(EROWS, 128)
  dst2d = jnp.concatenate([dst, N + pad_i % PAD_ROWS]).reshape(EROWS, 128)

  k1 = gamma * lax.rsqrt(running_var + 1e-5)
  kc1 = jnp.stack([k1, (b1 - running_mean) * k1 + beta])
  kc2 = jnp.stack([jnp.ones((128,), f32), bc0])
  kc3 = jnp.stack([jnp.ones((128,), f32), bc1])
  w2p = jnp.zeros((128, 128), f32).at[:, :C].set(W2)
  b2p = jnp.full((1, 128), -1e30, f32).at[0, :C].set(b2)
  batch2d = batch.astype(jnp.int32).reshape(1, N)

  _deg_kernel, _agg_kernel = _sc_kernels()
  deg_parts = _deg_kernel(dst2d).T

  p1, dis = _tc(
      _prep_body,
      (jax.ShapeDtypeStruct((N, 128), f32), jax.ShapeDtypeStruct((N, 1), f32)),
      deg_parts, x, W1)

  zblk = jnp.zeros((ACC_N // NS, 128), jnp.float32)
  agg1 = _agg_kernel(zblk, p1, src2d, dst2d)
  p2 = _tc(_mid_body, jax.ShapeDtypeStruct((N, 128), f32),
           agg1, p1, dis, kc1, Wc0)
  agg2 = _agg_kernel(zblk, p2, src2d, dst2d)
  p3 = _tc(_mid_body, jax.ShapeDtypeStruct((N, 128), f32),
           agg2, p2, dis, kc2, Wc1)
  agg3 = _agg_kernel(zblk, p3, src2d, dst2d)
  out = _tc(_final_body, jax.ShapeDtypeStruct((G, C), f32),
            agg3, p3, dis, kc3[1:2], batch2d, w2p, b2p)
  return out


# speedup vs baseline: 1.2495x; 1.2495x over previous
import functools

import jax
import jax.numpy as jnp
from jax import lax
from jax.experimental import pallas as pl
from jax.experimental.pallas import tpu as pltpu
from jax.experimental.pallas import tpu_sc as plsc

N = 10000
E = 320000
D = 128
G = 64
C = 10

NC = 2
NS = 16
NW = NC * NS

EROWS = 2560
EP = EROWS * 128
ROWS_PER_W = EROWS // NW

PAD_ROWS = 240
ACC_N = N + PAD_ROWS


def _deg_body(dst_hbm, out_hbm, idx_v, hist_v):
  c = lax.axis_index("c")
  s = lax.axis_index("s")
  w = s * NC + c
  pltpu.sync_copy(dst_hbm.at[pl.ds(w * ROWS_PER_W, ROWS_PER_W)], idx_v)
  zeros16 = jnp.zeros((16,), jnp.float32)

  def zbody(i, carry):
    hist_v[pl.ds(i * 16, 16)] = zeros16
    return carry

  lax.fori_loop(0, ACC_N // 16, zbody, 0)
  ones16 = jnp.full((16,), 1.0, jnp.float32)

  def body(r, carry):
    for j in range(8):
      idx = idx_v[r, pl.ds(j * 16, 16)]
      plsc.addupdate_scatter(hist_v, [idx], ones16)
    return carry

  lax.fori_loop(0, ROWS_PER_W, body, 0)
  pltpu.sync_copy(hist_v, out_hbm.at[w])


def _agg_body(z_hbm, p_hbm, src_hbm, dst_hbm, out_hbm, acc, sblk, dblk, rows0,
              rows1, gsem0, gsem1):
  c = lax.axis_index("c")
  s = lax.axis_index("s")
  w = s * NC + c

  tile_rows = ACC_N // NS
  zbase = s * tile_rows
  pltpu.sync_copy(z_hbm, acc.at[pl.ds(zbase, tile_rows)])
  plsc.subcore_barrier()

  half = ROWS_PER_W // 2
  for h in range(2):
    base = w * ROWS_PER_W + h * half
    pltpu.sync_copy(src_hbm.at[pl.ds(base, half)], sblk)
    pltpu.sync_copy(dst_hbm.at[pl.ds(base, half)], dblk)
    pltpu.async_copy(p_hbm.at[sblk.at[0]], rows0, gsem0)
    pltpu.async_copy(p_hbm.at[sblk.at[1]], rows1, gsem1)

    def ebody(i, carry):
      c0 = 2 * i
      pltpu.make_async_copy(p_hbm.at[sblk.at[c0]], rows0, gsem0).wait()
      pltpu.sync_copy(rows0, acc.at[dblk.at[c0]], add=True)
      pltpu.async_copy(p_hbm.at[sblk.at[c0 + 2]], rows0, gsem0)
      pltpu.make_async_copy(p_hbm.at[sblk.at[c0 + 1]], rows1, gsem1).wait()
      pltpu.sync_copy(rows1, acc.at[dblk.at[c0 + 1]], add=True)
      pltpu.async_copy(p_hbm.at[sblk.at[c0 + 3]], rows1, gsem1)
      return carry

    lax.fori_loop(0, half // 2 - 1, ebody, 0)
    cl = half - 2
    pltpu.make_async_copy(p_hbm.at[sblk.at[cl]], rows0, gsem0).wait()
    pltpu.sync_copy(rows0, acc.at[dblk.at[cl]], add=True)
    pltpu.make_async_copy(p_hbm.at[sblk.at[cl + 1]], rows1, gsem1).wait()
    pltpu.sync_copy(rows1, acc.at[dblk.at[cl + 1]], add=True)
  plsc.subcore_barrier()

  pltpu.sync_copy(acc.at[pl.ds(zbase, tile_rows)],
                  out_hbm.at[c, pl.ds(zbase, tile_rows)])


@functools.cache
def _sc_kernels():
  mesh = plsc.VectorSubcoreMesh(core_axis_name="c", subcore_axis_name="s",
                                num_cores=NC, num_subcores=NS)
  deg = pl.kernel(
      _deg_body,
      out_type=jax.ShapeDtypeStruct((NW, ACC_N), jnp.float32),
      mesh=mesh,
      scratch_types=[
          pltpu.VMEM((ROWS_PER_W, 128), jnp.int32),
          pltpu.VMEM((ACC_N,), jnp.float32),
      ],
      compiler_params=pltpu.CompilerParams(needs_layout_passes=False),
  )
  agg = pl.kernel(
      _agg_body,
      out_type=jax.ShapeDtypeStruct((NC, ACC_N, 128), jnp.float32),
      mesh=mesh,
      scratch_types=[
          pltpu.VMEM_SHARED((ACC_N, 128), jnp.float32),
          pltpu.VMEM((ROWS_PER_W // 2, 128), jnp.int32),
          pltpu.VMEM((ROWS_PER_W // 2, 128), jnp.int32),
          pltpu.VMEM((128, 128), jnp.float32),
          pltpu.VMEM((128, 128), jnp.float32),
          pltpu.SemaphoreType.DMA,
          pltpu.SemaphoreType.DMA,
      ],
  )
  return deg, agg


def _prep_body(parts_ref, x_ref, w1_ref, p_ref, dis_ref):
  deg = jnp.sum(parts_ref[...], axis=1, keepdims=True)[:N] + 1.0
  dis = lax.rsqrt(deg)
  h = jnp.dot(x_ref[...], w1_ref[...], preferred_element_type=jnp.float32,
              precision=lax.Precision.HIGHEST)
  p_ref[...] = h * dis
  dis_ref[...] = dis


def _mid_body(parts_ref, p_ref, dis_ref, kc_ref, w_ref, pn_ref):
  dis = dis_ref[...]
  agg = p_ref[...] + parts_ref[0][:N] + parts_ref[1][:N]
  t = jnp.maximum(dis * agg * kc_ref[0:1] + kc_ref[1:2], 0.0)
  h = jnp.dot(t, w_ref[...], preferred_element_type=jnp.float32,
              precision=lax.Precision.HIGHEST)
  pn_ref[...] = h * dis


def _final_body(parts_ref, p_ref, dis_ref, cc_ref, batch_ref, w2_ref, b2_ref,
                out_ref):
  dis = dis_ref[...]
  agg = p_ref[...] + parts_ref[0][:N] + parts_ref[1][:N]
  h = jnp.maximum(dis * agg + cc_ref[...], 0.0)
  gids = lax.broadcasted_iota(jnp.int32, (G, N), 0)
  mask = (gids == batch_ref[...]).astype(jnp.float32)
  sums = jnp.dot(mask, h, preferred_element_type=jnp.float32,
                 precision=lax.Precision.HIGHEST)
  counts = jnp.sum(mask, axis=1, keepdims=True)
  pooled = sums / jnp.maximum(counts, 1.0)
  logits = jnp.dot(pooled, w2_ref[...], preferred_element_type=jnp.float32,
                   precision=lax.Precision.HIGHEST) + b2_ref[...]
  m = jnp.max(logits, axis=1, keepdims=True)
  e = jnp.exp(logits - m)
  lse = jnp.log(jnp.sum(e, axis=1, keepdims=True))
  out_ref[...] = (logits - m - lse)[:, :C]


def _tc(body, out_shape, *args):
  return pl.pallas_call(body, out_shape=out_shape)(*args)


def kernel(x, edge_index, batch, W1, b1, gamma, beta, running_mean, running_var,
           Wc0, bc0, Wc1, bc1, W2, b2):
  f32 = jnp.float32
  src = edge_index[0].astype(jnp.int32)
  dst = edge_index[1].astype(jnp.int32)
  pad = EP - E
  pad_i = jnp.arange(pad, dtype=jnp.int32)
  src2d = jnp.concatenate([src, pad_i % N]).reshape(EROWS, 128)
  dst2d = jnp.concatenate([dst, N + pad_i % PAD_ROWS]).reshape(EROWS, 128)

  k1 = gamma * lax.rsqrt(running_var + 1e-5)
  kc1 = jnp.stack([k1, (b1 - running_mean) * k1 + beta])
  kc2 = jnp.stack([jnp.ones((128,), f32), bc0])
  kc3 = jnp.stack([jnp.ones((128,), f32), bc1])
  w2p = jnp.zeros((128, 128), f32).at[:, :C].set(W2)
  b2p = jnp.full((1, 128), -1e30, f32).at[0, :C].set(b2)
  batch2d = batch.astype(jnp.int32).reshape(1, N)

  _deg_kernel, _agg_kernel = _sc_kernels()
  deg_parts = _deg_kernel(dst2d).T

  p1, dis = _tc(
      _prep_body,
      (jax.ShapeDtypeStruct((N, 128), f32), jax.ShapeDtypeStruct((N, 1), f32)),
      deg_parts, x, W1)

  zblk = jnp.zeros((ACC_N // NS, 128), jnp.float32)
  agg1 = _agg_kernel(zblk, p1, src2d, dst2d)
  p2 = _tc(_mid_body, jax.ShapeDtypeStruct((N, 128), f32),
           agg1, p1, dis, kc1, Wc0)
  agg2 = _agg_kernel(zblk, p2, src2d, dst2d)
  p3 = _tc(_mid_body, jax.ShapeDtypeStruct((N, 128), f32),
           agg2, p2, dis, kc2, Wc1)
  agg3 = _agg_kernel(zblk, p3, src2d, dst2d)
  out = _tc(_final_body, jax.ShapeDtypeStruct((G, C), f32),
            agg3, p3, dis, kc3[1:2], batch2d, w2p, b2p)
  return out


# speedup vs baseline: 1.2951x; 1.0365x over previous
import functools

import jax
import jax.numpy as jnp
from jax import lax
from jax.experimental import pallas as pl
from jax.experimental.pallas import tpu as pltpu
from jax.experimental.pallas import tpu_sc as plsc

N = 10000
E = 320000
D = 128
G = 64
C = 10

NC = 2
NS = 16
NW = NC * NS

EROWS = 2560
EP = EROWS * 128
ROWS_PER_W = EROWS // NW

PAD_ROWS = 240
ACC_N = N + PAD_ROWS


def _deg_body(dst_hbm, out_hbm, idx_v, hist_v):
  c = lax.axis_index("c")
  s = lax.axis_index("s")
  w = s * NC + c
  pltpu.sync_copy(dst_hbm.at[pl.ds(w * ROWS_PER_W, ROWS_PER_W)], idx_v)
  zeros16 = jnp.zeros((16,), jnp.float32)

  def zbody(i, carry):
    hist_v[pl.ds(i * 16, 16)] = zeros16
    return carry

  lax.fori_loop(0, ACC_N // 16, zbody, 0)
  ones16 = jnp.full((16,), 1.0, jnp.float32)

  def body(r, carry):
    for j in range(8):
      idx = idx_v[r, pl.ds(j * 16, 16)]
      plsc.addupdate_scatter(hist_v, [idx], ones16)
    return carry

  lax.fori_loop(0, ROWS_PER_W, body, 0)
  pltpu.sync_copy(hist_v, out_hbm.at[w])


def _agg_body(p_hbm, src_hbm, dst_hbm, out_hbm, acc, sblk, dblk, rows0,
              rows1, gsem0, gsem1):
  c = lax.axis_index("c")
  s = lax.axis_index("s")
  w = s * NC + c

  zeros16 = jnp.zeros((16,), jnp.float32)

  def zbody(i, carry):
    for j in range(8):
      rows0[i, pl.ds(j * 16, 16)] = zeros16
    return carry

  lax.fori_loop(0, 128, zbody, 0)
  tile_rows = ACC_N // NS
  zbase = s * tile_rows
  for k in range(tile_rows // 128):
    pltpu.sync_copy(rows0, acc.at[pl.ds(zbase + k * 128, 128)])
  plsc.subcore_barrier()

  half = ROWS_PER_W // 2
  for h in range(2):
    base = w * ROWS_PER_W + h * half
    pltpu.sync_copy(src_hbm.at[pl.ds(base, half)], sblk)
    pltpu.sync_copy(dst_hbm.at[pl.ds(base, half)], dblk)
    pltpu.async_copy(p_hbm.at[sblk.at[0]], rows0, gsem0)
    pltpu.async_copy(p_hbm.at[sblk.at[1]], rows1, gsem1)

    def ebody(i, carry):
      c0 = 2 * i
      pltpu.make_async_copy(p_hbm.at[sblk.at[c0]], rows0, gsem0).wait()
      pltpu.sync_copy(rows0, acc.at[dblk.at[c0]], add=True)
      pltpu.async_copy(p_hbm.at[sblk.at[c0 + 2]], rows0, gsem0)
      pltpu.make_async_copy(p_hbm.at[sblk.at[c0 + 1]], rows1, gsem1).wait()
      pltpu.sync_copy(rows1, acc.at[dblk.at[c0 + 1]], add=True)
      pltpu.async_copy(p_hbm.at[sblk.at[c0 + 3]], rows1, gsem1)
      return carry

    lax.fori_loop(0, half // 2 - 1, ebody, 0)
    cl = half - 2
    pltpu.make_async_copy(p_hbm.at[sblk.at[cl]], rows0, gsem0).wait()
    pltpu.sync_copy(rows0, acc.at[dblk.at[cl]], add=True)
    pltpu.make_async_copy(p_hbm.at[sblk.at[cl + 1]], rows1, gsem1).wait()
    pltpu.sync_copy(rows1, acc.at[dblk.at[cl + 1]], add=True)
  plsc.subcore_barrier()

  pltpu.sync_copy(acc.at[pl.ds(zbase, tile_rows)],
                  out_hbm.at[c, pl.ds(zbase, tile_rows)])


@functools.cache
def _sc_kernels():
  mesh = plsc.VectorSubcoreMesh(core_axis_name="c", subcore_axis_name="s",
                                num_cores=NC, num_subcores=NS)
  deg = pl.kernel(
      _deg_body,
      out_type=jax.ShapeDtypeStruct((NW, ACC_N), jnp.float32),
      mesh=mesh,
      scratch_types=[
          pltpu.VMEM((ROWS_PER_W, 128), jnp.int32),
          pltpu.VMEM((ACC_N,), jnp.float32),
      ],
      compiler_params=pltpu.CompilerParams(needs_layout_passes=False),
  )
  agg = pl.kernel(
      _agg_body,
      out_type=jax.ShapeDtypeStruct((NC, ACC_N, 128), jnp.float32),
      mesh=mesh,
      scratch_types=[
          pltpu.VMEM_SHARED((ACC_N, 128), jnp.float32),
          pltpu.VMEM((ROWS_PER_W // 2, 128), jnp.int32),
          pltpu.VMEM((ROWS_PER_W // 2, 128), jnp.int32),
          pltpu.VMEM((128, 128), jnp.float32),
          pltpu.VMEM((128, 128), jnp.float32),
          pltpu.SemaphoreType.DMA,
          pltpu.SemaphoreType.DMA,
      ],
  )
  return deg, agg


def _mm_body(x_ref, w1_ref, h_ref):
  h_ref[...] = jnp.dot(x_ref[...], w1_ref[...],
                       preferred_element_type=jnp.float32,
                       precision=lax.Precision.HIGHEST)


def _prep_body(parts_ref, h_ref, p_ref, dis_ref):
  deg = jnp.sum(parts_ref[...], axis=1, keepdims=True)[:N] + 1.0
  dis = lax.rsqrt(deg)
  p_ref[...] = h_ref[...] * dis
  dis_ref[...] = dis


def _mid_body(parts_ref, p_ref, dis_ref, kc_ref, w_ref, pn_ref):
  dis = dis_ref[...]
  agg = p_ref[...] + parts_ref[0][:N] + parts_ref[1][:N]
  t = jnp.maximum(dis * agg * kc_ref[0:1] + kc_ref[1:2], 0.0)
  h = jnp.dot(t, w_ref[...], preferred_element_type=jnp.float32,
              precision=lax.Precision.HIGHEST)
  pn_ref[...] = h * dis


def _final_body(parts_ref, p_ref, dis_ref, cc_ref, batch_ref, w2_ref, b2_ref,
                out_ref):
  dis = dis_ref[...]
  agg = p_ref[...] + parts_ref[0][:N] + parts_ref[1][:N]
  h = jnp.maximum(dis * agg + cc_ref[...], 0.0)
  gids = lax.broadcasted_iota(jnp.int32, (G, N), 0)
  mask = (gids == batch_ref[...]).astype(jnp.float32)
  sums = jnp.dot(mask, h, preferred_element_type=jnp.float32,
                 precision=lax.Precision.HIGHEST)
  counts = jnp.sum(mask, axis=1, keepdims=True)
  pooled = sums / jnp.maximum(counts, 1.0)
  logits = jnp.dot(pooled, w2_ref[...], preferred_element_type=jnp.float32,
                   precision=lax.Precision.HIGHEST) + b2_ref[...]
  m = jnp.max(logits, axis=1, keepdims=True)
  e = jnp.exp(logits - m)
  lse = jnp.log(jnp.sum(e, axis=1, keepdims=True))
  out_ref[...] = (logits - m - lse)[:, :C]


def _tc(body, out_shape, *args):
  return pl.pallas_call(body, out_shape=out_shape)(*args)


def kernel(x, edge_index, batch, W1, b1, gamma, beta, running_mean, running_var,
           Wc0, bc0, Wc1, bc1, W2, b2):
  f32 = jnp.float32
  src = edge_index[0].astype(jnp.int32)
  dst = edge_index[1].astype(jnp.int32)
  pad = EP - E
  pad_i = jnp.arange(pad, dtype=jnp.int32)
  src2d = jnp.concatenate([src, pad_i % N]).reshape(EROWS, 128)
  dst2d = jnp.concatenate([dst, N + pad_i % PAD_ROWS]).reshape(EROWS, 128)

  k1 = gamma * lax.rsqrt(running_var + 1e-5)
  kc1 = jnp.stack([k1, (b1 - running_mean) * k1 + beta])
  kc2 = jnp.stack([jnp.ones((128,), f32), bc0])
  kc3 = jnp.stack([jnp.ones((128,), f32), bc1])
  w2p = jnp.zeros((128, 128), f32).at[:, :C].set(W2)
  b2p = jnp.full((1, 128), -1e30, f32).at[0, :C].set(b2)
  batch2d = batch.astype(jnp.int32).reshape(1, N)

  _deg_kernel, _agg_kernel = _sc_kernels()
  deg_parts = _deg_kernel(dst2d).T

  h1 = _tc(_mm_body, jax.ShapeDtypeStruct((N, 128), f32), x, W1)
  p1, dis = _tc(
      _prep_body,
      (jax.ShapeDtypeStruct((N, 128), f32), jax.ShapeDtypeStruct((N, 1), f32)),
      deg_parts, h1)

  agg1 = _agg_kernel(p1, src2d, dst2d)
  p2 = _tc(_mid_body, jax.ShapeDtypeStruct((N, 128), f32),
           agg1, p1, dis, kc1, Wc0)
  agg2 = _agg_kernel(p2, src2d, dst2d)
  p3 = _tc(_mid_body, jax.ShapeDtypeStruct((N, 128), f32),
           agg2, p2, dis, kc2, Wc1)
  agg3 = _agg_kernel(p3, src2d, dst2d)
  out = _tc(_final_body, jax.ShapeDtypeStruct((G, C), f32),
            agg3, p3, dis, kc3[1:2], batch2d, w2p, b2p)
  return out


# speedup vs baseline: 1.3077x; 1.0097x over previous
import functools

import jax
import jax.numpy as jnp
from jax import lax
from jax.experimental import pallas as pl
from jax.experimental.pallas import tpu as pltpu
from jax.experimental.pallas import tpu_sc as plsc

N = 10000
E = 320000
D = 128
G = 64
C = 10

NC = 2
NS = 16
NW = NC * NS

EROWS = 2560
EP = EROWS * 128
ROWS_PER_W = EROWS // NW

PAD_ROWS = 240
ACC_N = N + PAD_ROWS


def _deg_body(dst_hbm, out_hbm, idx_v, hist_v):
  c = lax.axis_index("c")
  s = lax.axis_index("s")
  w = s * NC + c
  pltpu.sync_copy(dst_hbm.at[pl.ds(w * ROWS_PER_W, ROWS_PER_W)], idx_v)
  zeros16 = jnp.zeros((16,), jnp.float32)

  def zbody(i, carry):
    hist_v[pl.ds(i * 16, 16)] = zeros16
    return carry

  lax.fori_loop(0, ACC_N // 16, zbody, 0)
  ones16 = jnp.full((16,), 1.0, jnp.float32)

  def body(r, carry):
    for j in range(8):
      idx = idx_v[r, pl.ds(j * 16, 16)]
      plsc.addupdate_scatter(hist_v, [idx], ones16)
    return carry

  lax.fori_loop(0, ROWS_PER_W, body, 0)
  pltpu.sync_copy(hist_v, out_hbm.at[w])


def _agg_body(p_hbm, src_hbm, dst_hbm, out_hbm, acc, sblk, dblk, rows0,
              rows1, gsem0, gsem1):
  c = lax.axis_index("c")
  s = lax.axis_index("s")
  w = s * NC + c

  zeros16 = jnp.zeros((16,), jnp.float32)

  def zbody(i, carry):
    for j in range(8):
      rows0[i, pl.ds(j * 16, 16)] = zeros16
    return carry

  lax.fori_loop(0, 128, zbody, 0)
  tile_rows = ACC_N // NS
  zbase = s * tile_rows
  for k in range(tile_rows // 128):
    pltpu.sync_copy(rows0, acc.at[pl.ds(zbase + k * 128, 128)])
  plsc.subcore_barrier()

  half = ROWS_PER_W // 2
  for h in range(2):
    base = w * ROWS_PER_W + h * half
    pltpu.sync_copy(src_hbm.at[pl.ds(base, half)], sblk)
    pltpu.sync_copy(dst_hbm.at[pl.ds(base, half)], dblk)
    pltpu.async_copy(p_hbm.at[sblk.at[0]], rows0, gsem0)
    pltpu.async_copy(p_hbm.at[sblk.at[1]], rows1, gsem1)

    def ebody(i, carry):
      c0 = 2 * i
      pltpu.make_async_copy(p_hbm.at[sblk.at[c0]], rows0, gsem0).wait()
      pltpu.sync_copy(rows0, acc.at[dblk.at[c0]], add=True)
      pltpu.async_copy(p_hbm.at[sblk.at[c0 + 2]], rows0, gsem0)
      pltpu.make_async_copy(p_hbm.at[sblk.at[c0 + 1]], rows1, gsem1).wait()
      pltpu.sync_copy(rows1, acc.at[dblk.at[c0 + 1]], add=True)
      pltpu.async_copy(p_hbm.at[sblk.at[c0 + 3]], rows1, gsem1)
      return carry

    lax.fori_loop(0, half // 2 - 1, ebody, 0)
    cl = half - 2
    pltpu.make_async_copy(p_hbm.at[sblk.at[cl]], rows0, gsem0).wait()
    pltpu.sync_copy(rows0, acc.at[dblk.at[cl]], add=True)
    pltpu.make_async_copy(p_hbm.at[sblk.at[cl + 1]], rows1, gsem1).wait()
    pltpu.sync_copy(rows1, acc.at[dblk.at[cl + 1]], add=True)
  plsc.subcore_barrier()

  pltpu.sync_copy(acc.at[pl.ds(zbase, tile_rows)],
                  out_hbm.at[c, pl.ds(zbase, tile_rows)])


@functools.cache
def _sc_kernels():
  mesh = plsc.VectorSubcoreMesh(core_axis_name="c", subcore_axis_name="s",
                                num_cores=NC, num_subcores=NS)
  deg = pl.kernel(
      _deg_body,
      out_type=jax.ShapeDtypeStruct((NW, ACC_N), jnp.float32),
      mesh=mesh,
      scratch_types=[
          pltpu.VMEM((ROWS_PER_W, 128), jnp.int32),
          pltpu.VMEM((ACC_N,), jnp.float32),
      ],
      compiler_params=pltpu.CompilerParams(needs_layout_passes=False),
  )
  agg = pl.kernel(
      _agg_body,
      out_type=jax.ShapeDtypeStruct((NC, ACC_N, 128), jnp.float32),
      mesh=mesh,
      scratch_types=[
          pltpu.VMEM_SHARED((ACC_N, 128), jnp.float32),
          pltpu.VMEM((ROWS_PER_W // 2, 128), jnp.int32),
          pltpu.VMEM((ROWS_PER_W // 2, 128), jnp.int32),
          pltpu.VMEM((128, 128), jnp.float32),
          pltpu.VMEM((128, 128), jnp.float32),
          pltpu.SemaphoreType.DMA,
          pltpu.SemaphoreType.DMA,
      ],
  )
  return deg, agg


def _mm_body(x_ref, w1_ref, h_ref):
  h_ref[...] = jnp.dot(x_ref[...], w1_ref[...],
                       preferred_element_type=jnp.float32,
                       precision=lax.Precision.HIGHEST)


def _prep_body(parts_ref, h_ref, p_ref, dis_ref):
  deg = jnp.sum(parts_ref[...], axis=1, keepdims=True) + 1.0
  dis = lax.rsqrt(deg)
  p_ref[...] = h_ref[...] * dis
  dis_ref[...] = dis


def _mid_body(parts_ref, p_ref, dis_ref, kc_ref, w_ref, pn_ref):
  dis = dis_ref[...]
  agg = p_ref[...] + parts_ref[0] + parts_ref[1]
  t = jnp.maximum(dis * agg * kc_ref[0:1] + kc_ref[1:2], 0.0)
  h = jnp.dot(t, w_ref[...], preferred_element_type=jnp.float32,
              precision=lax.Precision.HIGHEST)
  pn_ref[...] = h * dis


def _final_body(parts_ref, p_ref, dis_ref, cc_ref, batch_ref, w2_ref, b2_ref,
                out_ref):
  dis = dis_ref[...]
  agg = p_ref[...] + parts_ref[0][:N] + parts_ref[1][:N]
  h = jnp.maximum(dis * agg + cc_ref[...], 0.0)
  gids = lax.broadcasted_iota(jnp.int32, (G, N), 0)
  mask = (gids == batch_ref[...]).astype(jnp.float32)
  sums = jnp.dot(mask, h, preferred_element_type=jnp.float32,
                 precision=lax.Precision.HIGHEST)
  counts = jnp.sum(mask, axis=1, keepdims=True)
  pooled = sums / jnp.maximum(counts, 1.0)
  logits = jnp.dot(pooled, w2_ref[...], preferred_element_type=jnp.float32,
                   precision=lax.Precision.HIGHEST) + b2_ref[...]
  m = jnp.max(logits, axis=1, keepdims=True)
  e = jnp.exp(logits - m)
  lse = jnp.log(jnp.sum(e, axis=1, keepdims=True))
  out_ref[...] = (logits - m - lse)[:, :C]


def _tc(body, out_shape, *args):
  return pl.pallas_call(body, out_shape=out_shape)(*args)


BLK = 2000
NB = N // BLK

_row = lambda i: (i, 0)
_whole = lambda i: (0, 0)


def _mm_call(x, w1):
  return pl.pallas_call(
      _mm_body,
      out_shape=jax.ShapeDtypeStruct((N, 128), jnp.float32),
      grid=(NB,),
      in_specs=[pl.BlockSpec((BLK, 128), _row), pl.BlockSpec((128, 128), _whole)],
      out_specs=pl.BlockSpec((BLK, 128), _row),
  )(x, w1)


def _prep_call(parts, h):
  return pl.pallas_call(
      _prep_body,
      out_shape=(jax.ShapeDtypeStruct((N, 128), jnp.float32),
                 jax.ShapeDtypeStruct((N, 1), jnp.float32)),
      grid=(NB,),
      in_specs=[pl.BlockSpec((BLK, NW), _row), pl.BlockSpec((BLK, 128), _row)],
      out_specs=(pl.BlockSpec((BLK, 128), _row), pl.BlockSpec((BLK, 1), _row)),
  )(parts, h)


def _mid_call(parts, p, dis, kc, w):
  return pl.pallas_call(
      _mid_body,
      out_shape=jax.ShapeDtypeStruct((N, 128), jnp.float32),
      grid=(NB,),
      in_specs=[
          pl.BlockSpec((2, BLK, 128), lambda i: (0, i, 0)),
          pl.BlockSpec((BLK, 128), _row),
          pl.BlockSpec((BLK, 1), _row),
          pl.BlockSpec((2, 128), _whole),
          pl.BlockSpec((128, 128), _whole),
      ],
      out_specs=pl.BlockSpec((BLK, 128), _row),
  )(parts, p, dis, kc, w)


def kernel(x, edge_index, batch, W1, b1, gamma, beta, running_mean, running_var,
           Wc0, bc0, Wc1, bc1, W2, b2):
  f32 = jnp.float32
  src = edge_index[0].astype(jnp.int32)
  dst = edge_index[1].astype(jnp.int32)
  pad = EP - E
  pad_i = jnp.arange(pad, dtype=jnp.int32)
  src2d = jnp.concatenate([src, pad_i % N]).reshape(EROWS, 128)
  dst2d = jnp.concatenate([dst, N + pad_i % PAD_ROWS]).reshape(EROWS, 128)

  k1 = gamma * lax.rsqrt(running_var + 1e-5)
  kc1 = jnp.stack([k1, (b1 - running_mean) * k1 + beta])
  kc2 = jnp.stack([jnp.ones((128,), f32), bc0])
  kc3 = jnp.stack([jnp.ones((128,), f32), bc1])
  w2p = jnp.zeros((128, 128), f32).at[:, :C].set(W2)
  b2p = jnp.full((1, 128), -1e30, f32).at[0, :C].set(b2)
  batch2d = batch.astype(jnp.int32).reshape(1, N)

  _deg_kernel, _agg_kernel = _sc_kernels()
  deg_parts = _deg_kernel(dst2d).T

  h1 = _mm_call(x, W1)
  p1, dis = _prep_call(deg_parts, h1)

  agg1 = _agg_kernel(p1, src2d, dst2d)
  p2 = _mid_call(agg1, p1, dis, kc1, Wc0)
  agg2 = _agg_kernel(p2, src2d, dst2d)
  p3 = _mid_call(agg2, p2, dis, kc2, Wc1)
  agg3 = _agg_kernel(p3, src2d, dst2d)
  out = _tc(_final_body, jax.ShapeDtypeStruct((G, C), f32),
            agg3, p3, dis, kc3[1:2], batch2d, w2p, b2p)
  return out
